# baseline (speedup 1.0000x reference)
import functools

import jax
import jax.numpy as jnp
from jax.experimental import pallas as pl
from jax.experimental.pallas import tpu as pltpu

ITERS = 3
EPS = 1e-8
LN_EPS = 1e-5


def _ln(x, g, b):
    m = jnp.mean(x, axis=-1, keepdims=True)
    v = jnp.mean((x - m) ** 2, axis=-1, keepdims=True)
    return (x - m) * jax.lax.rsqrt(v + LN_EPS) * g + b


def _dot(a, b):
    return jnp.dot(a, b, preferred_element_type=jnp.float32)


def _dot_t(a, b):
    return jax.lax.dot_general(
        a, b, (((1,), (1,)), ((), ())), preferred_element_type=jnp.float32)


def _decoder_kernel(bB, f_ref, ling_ref, linb_ref, slots0_ref, lnsg_ref,
                    lnsb_ref, wq_ref, wk_ref, wv_ref, wih_ref, whh_ref,
                    bih_ref, bhh_ref, lnmg_ref, lnmb_ref, w1_ref, b1_ref,
                    w2_ref, b2_ref, wo_ref, bo_ref, out_ref, slots_out_ref):
    D = slots0_ref.shape[-1]
    scale = D ** -0.5
    s0 = slots0_ref[...]
    ling = ling_ref[...]
    linb = linb_ref[...]
    lnsg = lnsg_ref[...]
    lnsb = lnsb_ref[...]
    lnmg = lnmg_ref[...]
    lnmb = lnmb_ref[...]
    wq = wq_ref[...]
    wk = wk_ref[...]
    wv = wv_ref[...]
    wih = wih_ref[...]
    whh = whh_ref[...]
    bih = bih_ref[...]
    bhh = bhh_ref[...]
    w1 = w1_ref[...]
    b1 = b1_ref[...]
    w2 = w2_ref[...]
    b2 = b2_ref[...]
    wo = wo_ref[...]
    bo = bo_ref[...]

    def one_batch(b, carry):
        x = _ln(f_ref[b], ling, linb)
        k = _dot(x, wk)
        v = _dot(x, wv)
        slots = s0
        for _ in range(ITERS):
            prev = slots
            q = _dot(_ln(slots, lnsg, lnsb), wq)
            dots = _dot_t(q, k) * scale
            mx = jnp.max(dots, axis=0, keepdims=True)
            e = jnp.exp(dots - mx)
            attn = e / jnp.sum(e, axis=0, keepdims=True) + EPS
            attn = attn / jnp.sum(attn, axis=1, keepdims=True)
            updates = _dot(attn, v)
            gx = _dot_t(updates, wih) + bih
            gh = _dot_t(prev, whh) + bhh
            r = jax.nn.sigmoid(gx[:, :D] + gh[:, :D])
            z = jax.nn.sigmoid(gx[:, D:2 * D] + gh[:, D:2 * D])
            n = jnp.tanh(gx[:, 2 * D:] + r * gh[:, 2 * D:])
            slots = (1.0 - z) * n + z * prev
            h = jnp.maximum(_dot(_ln(slots, lnmg, lnmb), w1) + b1, 0.0)
            slots = slots + _dot(h, w2) + b2
        out_ref[b] = jnp.maximum(_dot(slots, wo) + bo, 0.0)
        slots_out_ref[b] = slots
        return carry

    jax.lax.fori_loop(0, bB, one_batch, 0)


def _call(features, ln_in_g, ln_in_b, slots_init, ln_s_g, ln_s_b,
          Wq, Wk, Wv, W_ih, W_hh, b_ih, b_hh,
          ln_m_g, ln_m_b, W1, b1, W2, b2, Wo, bo, interpret=False):
    B, N, E = features.shape
    S, D = slots_init.shape
    O = Wo.shape[-1]
    bB = 16
    grid = (B // bB,)

    row = lambda a: a.reshape(1, -1)
    full = lambda a: pl.BlockSpec(a.shape, lambda i: (0,) * a.ndim)
    weights = [row(ln_in_g), row(ln_in_b), slots_init, row(ln_s_g),
               row(ln_s_b), Wq, Wk, Wv, W_ih, W_hh, row(b_ih), row(b_hh),
               row(ln_m_g), row(ln_m_b), W1, row(b1), W2, row(b2),
               Wo, row(bo)]

    out, slots = pl.pallas_call(
        functools.partial(_decoder_kernel, bB),
        grid=grid,
        in_specs=[pl.BlockSpec((bB, N, E), lambda i: (i, 0, 0))]
                 + [full(w) for w in weights],
        out_specs=[pl.BlockSpec((bB, S, O), lambda i: (i, 0, 0)),
                   pl.BlockSpec((bB, S, D), lambda i: (i, 0, 0))],
        out_shape=[jax.ShapeDtypeStruct((B, S, O), jnp.float32),
                   jax.ShapeDtypeStruct((B, S, D), jnp.float32)],
        compiler_params=pltpu.CompilerParams(
            dimension_semantics=("parallel",),
            vmem_limit_bytes=50 * 1024 * 1024,
        ),
        name="slot_decoder",
        interpret=interpret,
    )(features, *weights)
    return (out, slots)


def kernel(features, ln_in_g, ln_in_b, slots_init, ln_s_g, ln_s_b,
           Wq, Wk, Wv, W_ih, W_hh, b_ih, b_hh,
           ln_m_g, ln_m_b, W1, b1, W2, b2, Wo, bo):
    return _call(features, ln_in_g, ln_in_b, slots_init, ln_s_g, ln_s_b,
                 Wq, Wk, Wv, W_ih, W_hh, b_ih, b_hh,
                 ln_m_g, ln_m_b, W1, b1, W2, b2, Wo, bo)


# speedup vs baseline: 4.7341x; 4.7341x over previous
import functools

import jax
import jax.numpy as jnp
from jax.experimental import pallas as pl
from jax.experimental.pallas import tpu as pltpu

ITERS = 3
EPS = 1e-8
LN_EPS = 1e-5
SP = 16


def _ln(x, g, b):
    m = jnp.mean(x, axis=-1, keepdims=True)
    v = jnp.mean((x - m) ** 2, axis=-1, keepdims=True)
    return (x - m) * jax.lax.rsqrt(v + LN_EPS) * g + b


def _dot(a, b):
    return jnp.dot(a, b, preferred_element_type=jnp.float32)


def _dot_t(a, b):
    return jax.lax.dot_general(
        a, b, (((1,), (1,)), ((), ())), preferred_element_type=jnp.float32)


def _decoder_kernel(bB, S, f_ref, ling_ref, linb_ref, s0_ref, lnsg_ref,
                    lnsb_ref, wq_ref, wkv_ref, wih_ref, whh_ref,
                    bih_ref, bhh_ref, lnmg_ref, lnmb_ref, w1_ref, b1_ref,
                    w2_ref, b2_ref, wo_ref, bo_ref, out_ref, slots_out_ref,
                    kv_ref):
    D = s0_ref.shape[-1]
    ling = ling_ref[...]
    linb = linb_ref[...]
    lnsg = lnsg_ref[...]
    lnsb = lnsb_ref[...]
    lnmg = lnmg_ref[...]
    lnmb = lnmb_ref[...]
    wq = wq_ref[...]
    wih = wih_ref[...]
    whh = whh_ref[...]
    bih = bih_ref[...]
    bhh = bhh_ref[...]
    w1 = w1_ref[...]
    b1 = b1_ref[...]
    w2 = w2_ref[...]
    b2 = b2_ref[...]

    wkv = wkv_ref[...]
    for b in range(bB):
        x = _ln(f_ref[b], ling, linb)
        kv_ref[b] = _dot(x, wkv)

    slots = s0_ref[...]
    padmask = jax.lax.broadcasted_iota(jnp.int32, (1, SP, 1), 1) >= S

    for _ in range(ITERS):
        prev = slots
        q = _dot(_ln(slots, lnsg, lnsb), wq)
        dots = jnp.stack(
            [_dot_t(q[b * SP:(b + 1) * SP], kv_ref[b]) for b in range(bB)],
            axis=0)
        dots = jnp.where(padmask, -1e30, dots)
        mx = jnp.max(dots, axis=1, keepdims=True)
        e = jnp.exp(dots - mx)
        attn = e / jnp.sum(e, axis=1, keepdims=True) + EPS
        attn = attn / jnp.sum(attn, axis=-1, keepdims=True)
        updates = jnp.concatenate(
            [_dot(attn[b], kv_ref[b]) for b in range(bB)],
            axis=0)
        gx = _dot_t(updates, wih) + bih
        gh = _dot_t(prev, whh) + bhh
        r = jax.nn.sigmoid(gx[:, :D] + gh[:, :D])
        z = jax.nn.sigmoid(gx[:, D:2 * D] + gh[:, D:2 * D])
        n = jnp.tanh(gx[:, 2 * D:] + r * gh[:, 2 * D:])
        slots = (1.0 - z) * n + z * prev
        h = jnp.maximum(_dot(_ln(slots, lnmg, lnmb), w1) + b1, 0.0)
        slots = slots + _dot(h, w2) + b2

    out = jnp.maximum(_dot(slots, wo_ref[...]) + bo_ref[...], 0.0)
    for b in range(bB):
        out_ref[b] = out[b * SP:b * SP + S]
        slots_out_ref[b] = slots[b * SP:b * SP + S]


def _call(features, ln_in_g, ln_in_b, slots_init, ln_s_g, ln_s_b,
          Wq, Wk, Wv, W_ih, W_hh, b_ih, b_hh,
          ln_m_g, ln_m_b, W1, b1, W2, b2, Wo, bo, interpret=False):
    B, N, E = features.shape
    S, D = slots_init.shape
    O = Wo.shape[-1]
    bB = 16
    grid = (B // bB,)
    scale = D ** -0.5

    zD = jnp.zeros((D, D), jnp.float32)
    wq_pad = jnp.concatenate([Wq * scale, zD], axis=1)
    wkv = jnp.concatenate([Wk, Wv], axis=1)
    wih_pad = jnp.concatenate(
        [jnp.zeros((3 * D, D), jnp.float32), W_ih], axis=1)
    s0 = jnp.zeros((SP, D), jnp.float32).at[:S].set(slots_init)
    s0_flat = jnp.tile(s0, (bB, 1))

    row = lambda a: a.reshape(1, -1)
    full = lambda a: pl.BlockSpec(a.shape, lambda i: (0,) * a.ndim)
    weights = [row(ln_in_g), row(ln_in_b), s0_flat, row(ln_s_g),
               row(ln_s_b), wq_pad, wkv, wih_pad, W_hh, row(b_ih),
               row(b_hh), row(ln_m_g), row(ln_m_b), W1, row(b1), W2,
               row(b2), Wo, row(bo)]

    out, slots = pl.pallas_call(
        functools.partial(_decoder_kernel, bB, S),
        grid=grid,
        in_specs=[pl.BlockSpec((bB, N, E), lambda i: (i, 0, 0))]
                 + [full(w) for w in weights],
        out_specs=[pl.BlockSpec((bB, S, O), lambda i: (i, 0, 0)),
                   pl.BlockSpec((bB, S, D), lambda i: (i, 0, 0))],
        out_shape=[jax.ShapeDtypeStruct((B, S, O), jnp.float32),
                   jax.ShapeDtypeStruct((B, S, D), jnp.float32)],
        scratch_shapes=[pltpu.VMEM((bB, N, 2 * D), jnp.float32)],
        compiler_params=pltpu.CompilerParams(
            dimension_semantics=("parallel",),
            vmem_limit_bytes=50 * 1024 * 1024,
        ),
        name="slot_decoder",
        interpret=interpret,
    )(features, *weights)
    return (out, slots)


def kernel(features, ln_in_g, ln_in_b, slots_init, ln_s_g, ln_s_b,
           Wq, Wk, Wv, W_ih, W_hh, b_ih, b_hh,
           ln_m_g, ln_m_b, W1, b1, W2, b2, Wo, bo):
    return _call(features, ln_in_g, ln_in_b, slots_init, ln_s_g, ln_s_b,
                 Wq, Wk, Wv, W_ih, W_hh, b_ih, b_hh,
                 ln_m_g, ln_m_b, W1, b1, W2, b2, Wo, bo)


# speedup vs baseline: 5.2537x; 1.1098x over previous
import functools

import jax
import jax.numpy as jnp
from jax.experimental import pallas as pl
from jax.experimental.pallas import tpu as pltpu

ITERS = 3
EPS = 1e-8
LN_EPS = 1e-5
SP = 16


def _rowstats(x):
    m = jnp.mean(x, axis=-1, keepdims=True)
    s2 = jnp.mean(x * x, axis=-1, keepdims=True)
    r = jax.lax.rsqrt(s2 - m * m + LN_EPS)
    return m, r


def _dot(a, b):
    return jnp.dot(a, b, preferred_element_type=jnp.float32)


def _dot_t(a, b):
    return jax.lax.dot_general(
        a, b, (((1,), (1,)), ((), ())), preferred_element_type=jnp.float32)


def _decoder_kernel(bB, S, f_ref, wkvg_ref, ckvg_ref, ckvb_ref, s0_ref,
                    wqg_ref, cqg_ref, cqb_ref, wih_ref, whh_ref,
                    bih_ref, bhh_ref, w1g_ref, c1g_ref, c1b_ref,
                    w2_ref, b2_ref, wo_ref, bo_ref, out_ref, slots_out_ref,
                    kv_ref):
    D = s0_ref.shape[-1]
    wqg = wqg_ref[...]
    cqg = cqg_ref[...]
    cqb = cqb_ref[...]
    wih = wih_ref[...]
    whh = whh_ref[...]
    bih = bih_ref[...]
    bhh = bhh_ref[...]
    w1g = w1g_ref[...]
    c1g = c1g_ref[...]
    c1b = c1b_ref[...]
    w2 = w2_ref[...]
    b2 = b2_ref[...]

    wkvg = wkvg_ref[...]
    ckvg = ckvg_ref[...]
    ckvb = ckvb_ref[...]
    for b in range(bB):
        x = f_ref[b]
        m, r = _rowstats(x)
        raw = _dot(x.astype(jnp.bfloat16), wkvg)
        kv_ref[b] = (r * raw - (m * r) * ckvg + ckvb).astype(jnp.bfloat16)

    padmask = jax.lax.broadcasted_iota(jnp.int32, (1, SP, 1), 1) >= S
    wo = wo_ref[...]
    bo = bo_ref[...]

    def half(b0, nb):
        slots = s0_ref[b0 * SP:(b0 + nb) * SP]
        for _ in range(ITERS):
            prev = slots
            m, r = _rowstats(slots)
            q = r * _dot(slots, wqg) - (m * r) * cqg + cqb
            qb = q.astype(jnp.bfloat16)
            dots = jnp.stack(
                [_dot_t(qb[b * SP:(b + 1) * SP], kv_ref[b0 + b])
                 for b in range(nb)], axis=0)
            dots = jnp.where(padmask, -1e30, dots)
            mx = jnp.max(dots, axis=1, keepdims=True)
            e = jnp.exp(dots - mx)
            attn = e / jnp.sum(e, axis=1, keepdims=True) + EPS
            attn = attn / jnp.sum(attn, axis=-1, keepdims=True)
            attn16 = attn.astype(jnp.bfloat16)
            updates = jnp.concatenate(
                [_dot(attn16[b], kv_ref[b0 + b]) for b in range(nb)],
                axis=0)
            gx = _dot_t(updates, wih) + bih
            gh = _dot_t(prev, whh) + bhh
            r_ = jax.nn.sigmoid(gx[:, :D] + gh[:, :D])
            z = jax.nn.sigmoid(gx[:, D:2 * D] + gh[:, D:2 * D])
            n = jnp.tanh(gx[:, 2 * D:] + r_ * gh[:, 2 * D:])
            slots = (1.0 - z) * n + z * prev
            m2, r2 = _rowstats(slots)
            h = jnp.maximum(
                r2 * _dot(slots, w1g) - (m2 * r2) * c1g + c1b, 0.0)
            slots = slots + _dot(h, w2) + b2

        out = jnp.maximum(_dot(slots, wo) + bo, 0.0)
        for b in range(nb):
            out_ref[b0 + b] = out[b * SP:b * SP + S]
            slots_out_ref[b0 + b] = slots[b * SP:b * SP + S]

    half(0, bB)


def _call(features, ln_in_g, ln_in_b, slots_init, ln_s_g, ln_s_b,
          Wq, Wk, Wv, W_ih, W_hh, b_ih, b_hh,
          ln_m_g, ln_m_b, W1, b1, W2, b2, Wo, bo, interpret=False):
    B, N, E = features.shape
    S, D = slots_init.shape
    O = Wo.shape[-1]
    bB = 16
    grid = (B // bB,)
    scale = D ** -0.5

    f32 = jnp.float32
    row = lambda a: a.reshape(1, -1)
    wkv = jnp.concatenate([Wk, Wv], axis=1)
    wkvg = (ln_in_g[:, None] * wkv).astype(jnp.bfloat16)
    ckvg = row(ln_in_g @ wkv)
    ckvb = row(ln_in_b @ wkv)
    wq_s = Wq * scale
    zD = jnp.zeros((D, D), f32)
    wqg = jnp.concatenate([ln_s_g[:, None] * wq_s, zD], axis=1)
    z1 = jnp.zeros((1, D), f32)
    cqg = jnp.concatenate([row(ln_s_g @ wq_s), z1], axis=1)
    cqb = jnp.concatenate([row(ln_s_b @ wq_s), z1], axis=1)
    wih_pad = jnp.concatenate([jnp.zeros((3 * D, D), f32), W_ih], axis=1)
    w1g = ln_m_g[:, None] * W1
    c1g = row(ln_m_g @ W1)
    c1b = row(ln_m_b @ W1 + b1)
    s0 = jnp.zeros((SP, D), f32).at[:S].set(slots_init)
    s0_flat = jnp.tile(s0, (bB, 1))

    full = lambda a: pl.BlockSpec(a.shape, lambda i: (0,) * a.ndim)
    weights = [wkvg, ckvg, ckvb, s0_flat, wqg, cqg, cqb, wih_pad, W_hh,
               row(b_ih), row(b_hh), w1g, c1g, c1b, W2, row(b2),
               Wo, row(bo)]

    out, slots = pl.pallas_call(
        functools.partial(_decoder_kernel, bB, S),
        grid=grid,
        in_specs=[pl.BlockSpec((bB, N, E), lambda i: (i, 0, 0))]
                 + [full(w) for w in weights],
        out_specs=[pl.BlockSpec((bB, S, O), lambda i: (i, 0, 0)),
                   pl.BlockSpec((bB, S, D), lambda i: (i, 0, 0))],
        out_shape=[jax.ShapeDtypeStruct((B, S, O), f32),
                   jax.ShapeDtypeStruct((B, S, D), f32)],
        scratch_shapes=[pltpu.VMEM((bB, N, 2 * D), jnp.bfloat16)],
        compiler_params=pltpu.CompilerParams(
            dimension_semantics=("parallel",),
            vmem_limit_bytes=50 * 1024 * 1024,
        ),
        name="slot_decoder",
        interpret=interpret,
    )(features, *weights)
    return (out, slots)


def kernel(features, ln_in_g, ln_in_b, slots_init, ln_s_g, ln_s_b,
           Wq, Wk, Wv, W_ih, W_hh, b_ih, b_hh,
           ln_m_g, ln_m_b, W1, b1, W2, b2, Wo, bo):
    return _call(features, ln_in_g, ln_in_b, slots_init, ln_s_g, ln_s_b,
                 Wq, Wk, Wv, W_ih, W_hh, b_ih, b_hh,
                 ln_m_g, ln_m_b, W1, b1, W2, b2, Wo, bo)
